# baseline (speedup 1.0000x reference)
import jax
import jax.numpy as jnp
from jax import lax
from jax.experimental import pallas as pl
from jax.experimental.pallas import tpu as pltpu
from jax.experimental.pallas import tpu_sc as plsc

B, S, D, E, TOPK = 2, 2048, 1024, 8, 2
H = 4 * D
N = B * S
P = TOPK * N
BLK = 256
CB = N // BLK
NBLK = P // BLK + E
PMAX = NBLK * BLK
GRID_C = E * CB
TRASH = NBLK
NW = 32

_RT = 1024
_SQRT_HALF = 0.7071067811865476


def _router_body(x_ref, gw_ref, gb_ref, stril_ref,
                 eid_ref, rank_ref, wpair_ref, base16_ref, valid_ref,
                 xsidx_ref, outidx_ref, laux_ref,
                 cnt_ref, acc_ref):
    kk = pl.program_id(0)
    t = pl.program_id(1)
    first = (kk == 0) & (t == 0)
    last = (kk == 1) & (t == pl.num_programs(1) - 1)

    xf = x_ref[...]
    logits = jax.lax.dot_general(
        xf, gw_ref[...], (((1,), (1,)), ((), ())),
        preferred_element_type=jnp.float32) + gb_ref[...]

    lane = jax.lax.broadcasted_iota(jnp.int32, (_RT, E), 1)
    m1 = jnp.max(logits, axis=1, keepdims=True)
    i1 = jnp.min(jnp.where(logits == m1, lane, E), axis=1, keepdims=True)
    masked = jnp.where(lane == i1, -jnp.inf, logits)
    m2 = jnp.max(masked, axis=1, keepdims=True)
    i2 = jnp.min(jnp.where(masked == m2, lane, E), axis=1, keepdims=True)
    e2 = jnp.exp(m2 - m1)
    w1 = 1.0 / (1.0 + e2)
    w2 = e2 * w1

    sel_i = jnp.where(kk == 0, i1, i2)
    sel_w = jnp.where(kk == 0, w1, w2)
    M = (lane == sel_i).astype(jnp.float32)

    @pl.when(first)
    def _():
        cnt_ref[...] = jnp.zeros_like(cnt_ref)
        acc_ref[...] = jnp.zeros_like(acc_ref)

    excl = jax.lax.dot_general(
        stril_ref[...], M.astype(jnp.bfloat16), (((1,), (0,)), ((), ())),
        preferred_element_type=jnp.float32)
    rank_rows = excl + cnt_ref[...]
    rank_sel = jnp.sum(M * rank_rows, axis=1, keepdims=True)

    eid_ref[0] = sel_i
    rank_ref[0] = rank_sel.astype(jnp.int32)
    wpair_ref[0] = sel_w

    cnt_ref[...] += jnp.sum(M, axis=0, keepdims=True)

    @pl.when(kk == 0)
    def _():
        ex = jnp.exp(logits - m1)
        probs = ex / jnp.sum(ex, axis=1, keepdims=True)
        acc_ref[...] += jnp.sum(probs, axis=0, keepdims=True)

    @pl.when(last)
    def _():
        pm = acc_ref[...] / N
        laux_ref[...] = jnp.sum(pm * pm, keepdims=True).reshape(1, 1) * E

        cntv = cnt_ref[...]
        blk = jnp.floor((cntv + (BLK - 1)) / BLK)
        padded = blk * BLK
        ri = jax.lax.broadcasted_iota(jnp.int32, (E, E), 0)
        cj = jax.lax.broadcasted_iota(jnp.int32, (E, E), 1)
        upt = (ri <= cj).astype(jnp.float32)
        base_incl = jax.lax.dot_general(
            padded, upt, (((1,), (0,)), ((), ())),
            preferred_element_type=jnp.float32, precision=jax.lax.Precision.HIGHEST)
        base = base_incl - padded
        blk_incl = jax.lax.dot_general(
            blk, upt, (((1,), (0,)), ((), ())),
            preferred_element_type=jnp.float32, precision=jax.lax.Precision.HIGHEST)
        blkoff = blk_incl - blk
        base16_ref[...] = jnp.pad(base.astype(jnp.int32), ((0, 0), (0, 16 - E)))

        g128 = jax.lax.broadcasted_iota(jnp.int32, (1, GRID_C), 1)
        e_g = g128 >> 4
        cbv = (g128 & (CB - 1)).astype(jnp.float32)
        sel = (jax.lax.broadcasted_iota(jnp.int32, (E, GRID_C), 0)
               == jax.lax.broadcasted_iota(jnp.int32, (E, GRID_C), 1) // CB
               ).astype(jnp.float32)
        blk_g = jax.lax.dot_general(
            blk, sel, (((1,), (0,)), ((), ())),
            preferred_element_type=jnp.float32, precision=jax.lax.Precision.HIGHEST)
        blkoff_g = jax.lax.dot_general(
            blkoff, sel, (((1,), (0,)), ((), ())),
            preferred_element_type=jnp.float32, precision=jax.lax.Precision.HIGHEST)
        validv = (cbv < blk_g).astype(jnp.int32)
        xsidxv = jnp.maximum(blkoff_g + jnp.minimum(cbv, blk_g - 1),
                             0.0).astype(jnp.int32)
        valid_ref[...] = validv
        xsidx_ref[...] = xsidxv
        outidx_ref[...] = jnp.where(validv == 1, xsidxv, TRASH)


def _router(flat, gate_W, gate_b, stril):
    nt = N // _RT
    (eid, rank, wpair, base16, valid, xsidx, outidx,
     laux) = pl.pallas_call(
        _router_body,
        grid=(TOPK, nt),
        in_specs=[
            pl.BlockSpec((_RT, D), lambda k, t: (t, 0)),
            pl.BlockSpec((E, D), lambda k, t: (0, 0)),
            pl.BlockSpec((1, E), lambda k, t: (0, 0)),
            pl.BlockSpec((_RT, _RT), lambda k, t: (0, 0)),
        ],
        out_specs=[
            pl.BlockSpec((1, _RT, 1), lambda k, t: (k * (N // _RT) + t, 0, 0)),
            pl.BlockSpec((1, _RT, 1), lambda k, t: (k * (N // _RT) + t, 0, 0)),
            pl.BlockSpec((1, _RT, 1), lambda k, t: (k * (N // _RT) + t, 0, 0)),
            pl.BlockSpec((1, 16), lambda k, t: (0, 0)),
            pl.BlockSpec((1, GRID_C), lambda k, t: (0, 0)),
            pl.BlockSpec((1, GRID_C), lambda k, t: (0, 0)),
            pl.BlockSpec((1, GRID_C), lambda k, t: (0, 0)),
            pl.BlockSpec((1, 1), lambda k, t: (0, 0)),
        ],
        out_shape=[
            jax.ShapeDtypeStruct((TOPK * nt, _RT, 1), jnp.int32),
            jax.ShapeDtypeStruct((TOPK * nt, _RT, 1), jnp.int32),
            jax.ShapeDtypeStruct((TOPK * nt, _RT, 1), jnp.float32),
            jax.ShapeDtypeStruct((1, 16), jnp.int32),
            jax.ShapeDtypeStruct((1, GRID_C), jnp.int32),
            jax.ShapeDtypeStruct((1, GRID_C), jnp.int32),
            jax.ShapeDtypeStruct((1, GRID_C), jnp.int32),
            jax.ShapeDtypeStruct((1, 1), jnp.float32),
        ],
        scratch_shapes=[pltpu.VMEM((1, E), jnp.float32),
                        pltpu.VMEM((1, E), jnp.float32)],
        compiler_params=pltpu.CompilerParams(
            dimension_semantics=("arbitrary", "arbitrary"),
        ),
    )(flat, gate_W, gate_b.reshape(1, E), stril)
    return (eid.reshape(P), rank.reshape(P), wpair.reshape(P),
            base16.reshape(16), valid.reshape(GRID_C),
            xsidx.reshape(GRID_C), outidx.reshape(GRID_C), laux)


def _dispatch_body(x_hbm, eid_hbm, rank_hbm, w_hbm, base_hbm,
                   xs_hbm, wrow_hbm, pos_hbm,
                   base_v, eid_v, rank_v, w_v, pos_v,
                   pos2_v, tok2_v, rows_v, wbuf_v, sem):
    wid = lax.axis_index("s") * 2 + lax.axis_index("c")
    pbase = wid * (P // NW)

    pltpu.sync_copy(base_hbm, base_v)
    pltpu.sync_copy(eid_hbm.at[pl.ds(pbase, P // NW)], eid_v)
    pltpu.sync_copy(rank_hbm.at[pl.ds(pbase, P // NW)], rank_v)
    pltpu.sync_copy(w_hbm.at[pl.ds(pbase, P // NW)], w_v)

    def posloop(ci, carry):
        off = ci * 16
        ev = eid_v[pl.ds(off, 16)]
        rv = rank_v[pl.ds(off, 16)]
        bv = plsc.load_gather(base_v, [ev])
        posv = bv + rv
        pos_v[pl.ds(off, 16)] = posv
        pos2_v[ci, :] = posv
        tok2_v[ci, :] = (pbase + off + lax.iota(jnp.int32, 16)) & (N - 1)
        return carry

    lax.fori_loop(0, P // NW // 16, posloop, 0, unroll=False)
    pltpu.sync_copy(pos_v, pos_hbm.at[pl.ds(pbase, P // NW)])

    def chunk(ci, carry):
        off = ci * 16
        pltpu.async_copy(x_hbm.at[tok2_v.at[ci]], rows_v, sem).wait()
        pltpu.async_copy(rows_v, xs_hbm.at[pos2_v.at[ci]], sem).wait()
        wv = w_v[pl.ds(off, 16)]
        for j in range(16):
            wbuf_v[j, pl.ds(0, 16)] = wv[jnp.full((16,), j, jnp.int32)]
        pltpu.async_copy(wbuf_v, wrow_hbm.at[pos2_v.at[ci]], sem).wait()
        return carry

    lax.fori_loop(0, P // NW // 16, chunk, 0, unroll=False)


def _dispatch(x2d, eid, rank, wpair, base16):
    mesh = plsc.VectorSubcoreMesh(core_axis_name="c", subcore_axis_name="s")
    kfn = pl.kernel(
        _dispatch_body,
        compiler_params=pltpu.CompilerParams(needs_layout_passes=False),
        out_type=[
            jax.ShapeDtypeStruct((PMAX, D), jnp.float32),
            jax.ShapeDtypeStruct((PMAX, 128), jnp.float32),
            jax.ShapeDtypeStruct((P,), jnp.int32),
        ],
        mesh=mesh,
        scratch_types=[
            pltpu.VMEM((16,), jnp.int32),
            pltpu.VMEM((P // NW,), jnp.int32),
            pltpu.VMEM((P // NW,), jnp.int32),
            pltpu.VMEM((P // NW,), jnp.float32),
            pltpu.VMEM((P // NW,), jnp.int32),
            pltpu.VMEM((P // NW // 16, 16), jnp.int32),
            pltpu.VMEM((P // NW // 16, 16), jnp.int32),
            pltpu.VMEM((16, D), jnp.float32),
            pltpu.VMEM((16, 128), jnp.float32),
            pltpu.SemaphoreType.DMA,
        ],
    )
    return kfn(x2d, eid, rank, wpair, base16)


def _gffn_body(valid_ref, xsidx_ref, outidx_ref,
               xs_ref, w1_ref, b1_ref, w2_ref, b2_ref, wrow_ref, out_ref):
    g = pl.program_id(0)
    valid = valid_ref[g] == 1

    @pl.when(valid)
    def _():
        xb = xs_ref[...].astype(jnp.bfloat16)
        hpre = jax.lax.dot_general(
            xb, w1_ref[0], (((1,), (1,)), ((), ())),
            preferred_element_type=jnp.float32) + b1_ref[0]
        hact = hpre * 0.5 * (1.0 + jax.lax.erf(hpre * _SQRT_HALF))
        contrib = jax.lax.dot_general(
            hact.astype(jnp.bfloat16), w2_ref[0], (((1,), (1,)), ((), ())),
            preferred_element_type=jnp.float32) + b2_ref[0]
        wcol = wrow_ref[...][:, 0:1]
        out_ref[...] = contrib * wcol

    @pl.when(~valid)
    def _():
        out_ref[...] = jnp.zeros_like(out_ref)


def _gffn(xs, wrow, valid, xsidx, outidx, W1b, b1, W2b, b2):
    grid_spec = pltpu.PrefetchScalarGridSpec(
        num_scalar_prefetch=3,
        grid=(GRID_C,),
        in_specs=[
            pl.BlockSpec((BLK, D), lambda g, v, xi, oi: (xi[g], 0)),
            pl.BlockSpec((1, H, D), lambda g, v, xi, oi: (g // CB, 0, 0)),
            pl.BlockSpec((1, 1, H), lambda g, v, xi, oi: (g // CB, 0, 0)),
            pl.BlockSpec((1, D, H), lambda g, v, xi, oi: (g // CB, 0, 0)),
            pl.BlockSpec((1, 1, D), lambda g, v, xi, oi: (g // CB, 0, 0)),
            pl.BlockSpec((BLK, 128), lambda g, v, xi, oi: (xi[g], 0)),
        ],
        out_specs=pl.BlockSpec((BLK, D), lambda g, v, xi, oi: (oi[g], 0)),
    )
    return pl.pallas_call(
        _gffn_body,
        grid_spec=grid_spec,
        out_shape=jax.ShapeDtypeStruct(((NBLK + 1) * BLK, D), jnp.float32),
        compiler_params=pltpu.CompilerParams(
            dimension_semantics=("arbitrary",),
        ),
    )(valid, xsidx, outidx, xs, W1b, b1.reshape(E, 1, H), W2b,
      b2.reshape(E, 1, D), wrow)


def _combine_body(ys_hbm, p1_hbm, p2_hbm, out_hbm, p1_v, p2_v, buf_v, buf2_v,
                  sem):
    wid = lax.axis_index("s") * 2 + lax.axis_index("c")
    tbase = wid * (N // NW)

    pltpu.sync_copy(p1_hbm.at[pl.ds(tbase, N // NW)], p1_v)
    pltpu.sync_copy(p2_hbm.at[pl.ds(tbase, N // NW)], p2_v)

    def chunk(ci, carry):
        off = ci * 16
        i1 = p1_v[pl.ds(off, 16)]
        i2 = p2_v[pl.ds(off, 16)]
        cp1 = pltpu.async_copy(ys_hbm.at[i1], buf_v, sem)
        cp2 = pltpu.async_copy(ys_hbm.at[i2], buf2_v, sem)
        cp1.wait()
        cp2.wait()

        def addrow(j, carry2):
            for seg in range(D // 16):
                s = pl.ds(seg * 16, 16)
                buf_v[j, s] = buf_v[j, s] + buf2_v[j, s]
            return carry2

        lax.fori_loop(0, 16, addrow, 0, unroll=False)
        pltpu.sync_copy(buf_v, out_hbm.at[pl.ds(tbase + off, 16)])
        return carry

    lax.fori_loop(0, N // NW // 16, chunk, 0, unroll=False)


def _combine(ys, pos1, pos2):
    mesh = plsc.VectorSubcoreMesh(core_axis_name="c", subcore_axis_name="s")
    kfn = pl.kernel(
        _combine_body,
        out_type=jax.ShapeDtypeStruct((N, D), jnp.float32),
        mesh=mesh,
        scratch_types=[
            pltpu.VMEM((N // NW,), jnp.int32),
            pltpu.VMEM((N // NW,), jnp.int32),
            pltpu.VMEM((16, D), jnp.float32),
            pltpu.VMEM((16, D), jnp.float32),
            pltpu.SemaphoreType.DMA,
        ],
    )
    return kfn(ys, pos1, pos2)


def kernel(x, gate_W, gate_b, W1, b1, W2, b2):
    flat = x.reshape(N, D)
    W1b = W1.astype(jnp.bfloat16)
    W2b = W2.astype(jnp.bfloat16)
    ii = jnp.arange(_RT, dtype=jnp.int32)
    stril = (ii[None, :] < ii[:, None]).astype(jnp.bfloat16)
    (eid, rank, wpair, base16, valid, xsidx, outidx,
     laux) = _router(flat, gate_W, gate_b, stril)
    xs, wrow, pos = _dispatch(flat, eid, rank, wpair, base16)
    ys = _gffn(xs, wrow, valid, xsidx, outidx, W1b, b1, W2b, b2)
    out_flat = _combine(ys, pos[:N], pos[N:])
    return out_flat.reshape(B, S, D), laux.reshape(())


# speedup vs baseline: 1.1106x; 1.1106x over previous
import jax
import jax.numpy as jnp
from jax import lax
from jax.experimental import pallas as pl
from jax.experimental.pallas import tpu as pltpu
from jax.experimental.pallas import tpu_sc as plsc

B, S, D, E, TOPK = 2, 2048, 1024, 8, 2
H = 4 * D
N = B * S
P = TOPK * N
BLK = 256
CB = N // BLK
NBLK = P // BLK + E
PMAX = NBLK * BLK
GRID_C = E * CB
TRASH = NBLK
NW = 32

_RT = 1024
_SQRT_HALF = 0.7071067811865476


def _router_body(x_ref, gw_ref, gb_ref, stril_ref,
                 eid_ref, rank_ref, wpair_ref, base16_ref, valid_ref,
                 xsidx_ref, outidx_ref, laux_ref,
                 cnt_ref, acc_ref):
    kk = pl.program_id(0)
    t = pl.program_id(1)
    first = (kk == 0) & (t == 0)
    last = (kk == 1) & (t == pl.num_programs(1) - 1)

    xf = x_ref[...]
    logits = jax.lax.dot_general(
        xf, gw_ref[...], (((1,), (1,)), ((), ())),
        preferred_element_type=jnp.float32) + gb_ref[...]

    lane = jax.lax.broadcasted_iota(jnp.int32, (_RT, E), 1)
    m1 = jnp.max(logits, axis=1, keepdims=True)
    i1 = jnp.min(jnp.where(logits == m1, lane, E), axis=1, keepdims=True)
    masked = jnp.where(lane == i1, -jnp.inf, logits)
    m2 = jnp.max(masked, axis=1, keepdims=True)
    i2 = jnp.min(jnp.where(masked == m2, lane, E), axis=1, keepdims=True)
    e2 = jnp.exp(m2 - m1)
    w1 = 1.0 / (1.0 + e2)
    w2 = e2 * w1

    sel_i = jnp.where(kk == 0, i1, i2)
    sel_w = jnp.where(kk == 0, w1, w2)
    M = (lane == sel_i).astype(jnp.float32)

    @pl.when(first)
    def _():
        cnt_ref[...] = jnp.zeros_like(cnt_ref)
        acc_ref[...] = jnp.zeros_like(acc_ref)

    excl = jax.lax.dot_general(
        stril_ref[...], M.astype(jnp.bfloat16), (((1,), (0,)), ((), ())),
        preferred_element_type=jnp.float32)
    rank_rows = excl + cnt_ref[...]
    rank_sel = jnp.sum(M * rank_rows, axis=1, keepdims=True)

    eid_ref[0] = sel_i
    rank_ref[0] = rank_sel.astype(jnp.int32)
    wpair_ref[0] = sel_w

    cnt_ref[...] += jnp.sum(M, axis=0, keepdims=True)

    @pl.when(kk == 0)
    def _():
        ex = jnp.exp(logits - m1)
        probs = ex / jnp.sum(ex, axis=1, keepdims=True)
        acc_ref[...] += jnp.sum(probs, axis=0, keepdims=True)

    @pl.when(last)
    def _():
        pm = acc_ref[...] / N
        laux_ref[...] = jnp.sum(pm * pm, keepdims=True).reshape(1, 1) * E

        cntv = cnt_ref[...]
        blk = jnp.floor((cntv + (BLK - 1)) / BLK)
        padded = blk * BLK
        ri = jax.lax.broadcasted_iota(jnp.int32, (E, E), 0)
        cj = jax.lax.broadcasted_iota(jnp.int32, (E, E), 1)
        upt = (ri <= cj).astype(jnp.float32)
        base_incl = jax.lax.dot_general(
            padded, upt, (((1,), (0,)), ((), ())),
            preferred_element_type=jnp.float32, precision=jax.lax.Precision.HIGHEST)
        base = base_incl - padded
        blk_incl = jax.lax.dot_general(
            blk, upt, (((1,), (0,)), ((), ())),
            preferred_element_type=jnp.float32, precision=jax.lax.Precision.HIGHEST)
        blkoff = blk_incl - blk
        base16_ref[...] = jnp.pad(base.astype(jnp.int32), ((0, 0), (0, 16 - E)))

        bvec = jax.lax.broadcasted_iota(jnp.int32, (1, GRID_C), 1)
        lane_e = jax.lax.broadcasted_iota(jnp.int32, (1, E), 1)
        e_of_b = jnp.zeros((1, GRID_C), jnp.int32)
        for e in range(E - 1):
            incl_e = jnp.sum(
                jnp.where(lane_e == e, blk_incl, 0.0), axis=1,
                keepdims=True).astype(jnp.int32)
            e_of_b = e_of_b + (bvec >= incl_e).astype(jnp.int32)
        nused = jnp.sum(
            jnp.where(lane_e == E - 1, blk_incl, 0.0), axis=1,
            keepdims=True).astype(jnp.int32)
        validv = (bvec < nused).astype(jnp.int32)
        valid_ref[...] = validv
        xsidx_ref[...] = jnp.minimum(e_of_b, E - 1)
        outidx_ref[...] = jnp.where(validv == 1, bvec, TRASH)


def _router(flat, gate_W, gate_b, stril):
    nt = N // _RT
    (eid, rank, wpair, base16, valid, xsidx, outidx,
     laux) = pl.pallas_call(
        _router_body,
        grid=(TOPK, nt),
        in_specs=[
            pl.BlockSpec((_RT, D), lambda k, t: (t, 0)),
            pl.BlockSpec((E, D), lambda k, t: (0, 0)),
            pl.BlockSpec((1, E), lambda k, t: (0, 0)),
            pl.BlockSpec((_RT, _RT), lambda k, t: (0, 0)),
        ],
        out_specs=[
            pl.BlockSpec((1, _RT, 1), lambda k, t: (k * (N // _RT) + t, 0, 0)),
            pl.BlockSpec((1, _RT, 1), lambda k, t: (k * (N // _RT) + t, 0, 0)),
            pl.BlockSpec((1, _RT, 1), lambda k, t: (k * (N // _RT) + t, 0, 0)),
            pl.BlockSpec((1, 16), lambda k, t: (0, 0)),
            pl.BlockSpec((1, GRID_C), lambda k, t: (0, 0)),
            pl.BlockSpec((1, GRID_C), lambda k, t: (0, 0)),
            pl.BlockSpec((1, GRID_C), lambda k, t: (0, 0)),
            pl.BlockSpec((1, 1), lambda k, t: (0, 0)),
        ],
        out_shape=[
            jax.ShapeDtypeStruct((TOPK * nt, _RT, 1), jnp.int32),
            jax.ShapeDtypeStruct((TOPK * nt, _RT, 1), jnp.int32),
            jax.ShapeDtypeStruct((TOPK * nt, _RT, 1), jnp.float32),
            jax.ShapeDtypeStruct((1, 16), jnp.int32),
            jax.ShapeDtypeStruct((1, GRID_C), jnp.int32),
            jax.ShapeDtypeStruct((1, GRID_C), jnp.int32),
            jax.ShapeDtypeStruct((1, GRID_C), jnp.int32),
            jax.ShapeDtypeStruct((1, 1), jnp.float32),
        ],
        scratch_shapes=[pltpu.VMEM((1, E), jnp.float32),
                        pltpu.VMEM((1, E), jnp.float32)],
        compiler_params=pltpu.CompilerParams(
            dimension_semantics=("arbitrary", "arbitrary"),
        ),
    )(flat, gate_W, gate_b.reshape(1, E), stril)
    return (eid.reshape(P), rank.reshape(P), wpair.reshape(P),
            base16.reshape(16), valid.reshape(GRID_C),
            xsidx.reshape(GRID_C), outidx.reshape(GRID_C), laux)


def _dispatch_body(x_hbm, eid_hbm, rank_hbm, w_hbm, base_hbm,
                   xs_hbm, wrow_hbm, pos_hbm,
                   base_v, eid_v, rank_v, w_v, pos_v,
                   pos2_v, tok2_v, rows_v, wbuf_v, sem):
    wid = lax.axis_index("s") * 2 + lax.axis_index("c")
    pbase = wid * (P // NW)

    pltpu.sync_copy(base_hbm, base_v)
    pltpu.sync_copy(eid_hbm.at[pl.ds(pbase, P // NW)], eid_v)
    pltpu.sync_copy(rank_hbm.at[pl.ds(pbase, P // NW)], rank_v)
    pltpu.sync_copy(w_hbm.at[pl.ds(pbase, P // NW)], w_v)

    def posloop(ci, carry):
        off = ci * 16
        ev = eid_v[pl.ds(off, 16)]
        rv = rank_v[pl.ds(off, 16)]
        bv = plsc.load_gather(base_v, [ev])
        posv = bv + rv
        pos_v[pl.ds(off, 16)] = posv
        pos2_v[ci, :] = posv
        tok2_v[ci, :] = (pbase + off + lax.iota(jnp.int32, 16)) & (N - 1)
        return carry

    lax.fori_loop(0, P // NW // 16, posloop, 0, unroll=False)
    pltpu.sync_copy(pos_v, pos_hbm.at[pl.ds(pbase, P // NW)])

    def chunk(ci, carry):
        off = ci * 16
        pltpu.async_copy(x_hbm.at[tok2_v.at[ci]], rows_v, sem).wait()
        pltpu.async_copy(rows_v, xs_hbm.at[pos2_v.at[ci]], sem).wait()
        wv = w_v[pl.ds(off, 16)]
        for j in range(16):
            wbuf_v[j, pl.ds(0, 16)] = wv[jnp.full((16,), j, jnp.int32)]
        pltpu.async_copy(wbuf_v, wrow_hbm.at[pos2_v.at[ci]], sem).wait()
        return carry

    lax.fori_loop(0, P // NW // 16, chunk, 0, unroll=False)


def _dispatch(x2d, eid, rank, wpair, base16):
    mesh = plsc.VectorSubcoreMesh(core_axis_name="c", subcore_axis_name="s")
    kfn = pl.kernel(
        _dispatch_body,
        compiler_params=pltpu.CompilerParams(needs_layout_passes=False),
        out_type=[
            jax.ShapeDtypeStruct((PMAX, D), jnp.float32),
            jax.ShapeDtypeStruct((PMAX, 128), jnp.float32),
            jax.ShapeDtypeStruct((P,), jnp.int32),
        ],
        mesh=mesh,
        scratch_types=[
            pltpu.VMEM((16,), jnp.int32),
            pltpu.VMEM((P // NW,), jnp.int32),
            pltpu.VMEM((P // NW,), jnp.int32),
            pltpu.VMEM((P // NW,), jnp.float32),
            pltpu.VMEM((P // NW,), jnp.int32),
            pltpu.VMEM((P // NW // 16, 16), jnp.int32),
            pltpu.VMEM((P // NW // 16, 16), jnp.int32),
            pltpu.VMEM((16, D), jnp.float32),
            pltpu.VMEM((16, 128), jnp.float32),
            pltpu.SemaphoreType.DMA,
        ],
    )
    return kfn(x2d, eid, rank, wpair, base16)


def _gffn_body(valid_ref, eofb_ref, outidx_ref,
               xs_ref, w1_ref, b1_ref, w2_ref, b2_ref, wrow_ref, out_ref):
    g = pl.program_id(0)
    valid = valid_ref[g] == 1

    @pl.when(valid)
    def _():
        xb = xs_ref[...].astype(jnp.bfloat16)
        hpre = jax.lax.dot_general(
            xb, w1_ref[0], (((1,), (1,)), ((), ())),
            preferred_element_type=jnp.float32) + b1_ref[0]
        hact = hpre * 0.5 * (1.0 + jax.lax.erf(hpre * _SQRT_HALF))
        contrib = jax.lax.dot_general(
            hact.astype(jnp.bfloat16), w2_ref[0], (((1,), (1,)), ((), ())),
            preferred_element_type=jnp.float32) + b2_ref[0]
        wcol = wrow_ref[...][:, 0:1]
        out_ref[...] = contrib * wcol

    @pl.when(~valid)
    def _():
        out_ref[...] = jnp.zeros_like(out_ref)


def _gffn(xs, wrow, valid, eofb, outidx, W1b, b1, W2b, b2):
    grid_spec = pltpu.PrefetchScalarGridSpec(
        num_scalar_prefetch=3,
        grid=(NBLK,),
        in_specs=[
            pl.BlockSpec((BLK, D), lambda g, v, eb, oi: (g, 0)),
            pl.BlockSpec((1, H, D), lambda g, v, eb, oi: (eb[g], 0, 0)),
            pl.BlockSpec((1, 1, H), lambda g, v, eb, oi: (eb[g], 0, 0)),
            pl.BlockSpec((1, D, H), lambda g, v, eb, oi: (eb[g], 0, 0)),
            pl.BlockSpec((1, 1, D), lambda g, v, eb, oi: (eb[g], 0, 0)),
            pl.BlockSpec((BLK, 128), lambda g, v, eb, oi: (g, 0)),
        ],
        out_specs=pl.BlockSpec((BLK, D), lambda g, v, eb, oi: (oi[g], 0)),
    )
    return pl.pallas_call(
        _gffn_body,
        grid_spec=grid_spec,
        out_shape=jax.ShapeDtypeStruct(((NBLK + 1) * BLK, D), jnp.float32),
        compiler_params=pltpu.CompilerParams(
            dimension_semantics=("arbitrary",),
        ),
    )(valid, eofb, outidx, xs, W1b, b1.reshape(E, 1, H), W2b,
      b2.reshape(E, 1, D), wrow)


def _combine_body(ys_hbm, p1_hbm, p2_hbm, out_hbm, p1_v, p2_v, buf_v, buf2_v,
                  sem):
    wid = lax.axis_index("s") * 2 + lax.axis_index("c")
    tbase = wid * (N // NW)

    pltpu.sync_copy(p1_hbm.at[pl.ds(tbase, N // NW)], p1_v)
    pltpu.sync_copy(p2_hbm.at[pl.ds(tbase, N // NW)], p2_v)

    def chunk(ci, carry):
        off = ci * 16
        i1 = p1_v[pl.ds(off, 16)]
        i2 = p2_v[pl.ds(off, 16)]
        cp1 = pltpu.async_copy(ys_hbm.at[i1], buf_v, sem)
        cp2 = pltpu.async_copy(ys_hbm.at[i2], buf2_v, sem)
        cp1.wait()
        cp2.wait()

        def addrow(j, carry2):
            for seg in range(D // 16):
                s = pl.ds(seg * 16, 16)
                buf_v[j, s] = buf_v[j, s] + buf2_v[j, s]
            return carry2

        lax.fori_loop(0, 16, addrow, 0, unroll=False)
        pltpu.sync_copy(buf_v, out_hbm.at[pl.ds(tbase + off, 16)])
        return carry

    lax.fori_loop(0, N // NW // 16, chunk, 0, unroll=False)


def _combine(ys, pos1, pos2):
    mesh = plsc.VectorSubcoreMesh(core_axis_name="c", subcore_axis_name="s")
    kfn = pl.kernel(
        _combine_body,
        out_type=jax.ShapeDtypeStruct((N, D), jnp.float32),
        mesh=mesh,
        scratch_types=[
            pltpu.VMEM((N // NW,), jnp.int32),
            pltpu.VMEM((N // NW,), jnp.int32),
            pltpu.VMEM((16, D), jnp.float32),
            pltpu.VMEM((16, D), jnp.float32),
            pltpu.SemaphoreType.DMA,
        ],
    )
    return kfn(ys, pos1, pos2)


def kernel(x, gate_W, gate_b, W1, b1, W2, b2):
    flat = x.reshape(N, D)
    W1b = W1.astype(jnp.bfloat16)
    W2b = W2.astype(jnp.bfloat16)
    ii = jnp.arange(_RT, dtype=jnp.int32)
    stril = (ii[None, :] < ii[:, None]).astype(jnp.bfloat16)
    (eid, rank, wpair, base16, valid, xsidx, outidx,
     laux) = _router(flat, gate_W, gate_b, stril)
    xs, wrow, pos = _dispatch(flat, eid, rank, wpair, base16)
    ys = _gffn(xs, wrow, valid, xsidx, outidx, W1b, b1, W2b, b2)
    out_flat = _combine(ys, pos[:N], pos[N:])
    return out_flat.reshape(B, S, D), laux.reshape(())


# speedup vs baseline: 1.2415x; 1.1179x over previous
import jax
import jax.numpy as jnp
from jax import lax
from jax.experimental import pallas as pl
from jax.experimental.pallas import tpu as pltpu
from jax.experimental.pallas import tpu_sc as plsc

B, S, D, E, TOPK = 2, 2048, 1024, 8, 2
H = 4 * D
N = B * S
P = TOPK * N
BLK = 256
CB = N // BLK
NBLK = P // BLK + E
PMAX = NBLK * BLK
GRID_C = E * CB
TRASH = NBLK
NW = 32

_RT = 1024
_SQRT_HALF = 0.7071067811865476


def _router_body(x_ref, gw_ref, gb_ref, stril_ref,
                 eid_ref, rank_ref, wpair_ref, base16_ref, valid_ref,
                 xsidx_ref, outidx_ref, laux_ref,
                 cnt_ref, acc_ref):
    kk = pl.program_id(0)
    t = pl.program_id(1)
    first = (kk == 0) & (t == 0)
    last = (kk == 1) & (t == pl.num_programs(1) - 1)

    xf = x_ref[...]
    logits = jax.lax.dot_general(
        xf, gw_ref[...], (((1,), (1,)), ((), ())),
        preferred_element_type=jnp.float32) + gb_ref[...]

    lane = jax.lax.broadcasted_iota(jnp.int32, (_RT, E), 1)
    m1 = jnp.max(logits, axis=1, keepdims=True)
    i1 = jnp.min(jnp.where(logits == m1, lane, E), axis=1, keepdims=True)
    masked = jnp.where(lane == i1, -jnp.inf, logits)
    m2 = jnp.max(masked, axis=1, keepdims=True)
    i2 = jnp.min(jnp.where(masked == m2, lane, E), axis=1, keepdims=True)
    e2 = jnp.exp(m2 - m1)
    w1 = 1.0 / (1.0 + e2)
    w2 = e2 * w1

    sel_i = jnp.where(kk == 0, i1, i2)
    sel_w = jnp.where(kk == 0, w1, w2)
    M = (lane == sel_i).astype(jnp.float32)

    @pl.when(first)
    def _():
        cnt_ref[...] = jnp.zeros_like(cnt_ref)
        acc_ref[...] = jnp.zeros_like(acc_ref)

    excl = jax.lax.dot_general(
        stril_ref[...], M.astype(jnp.bfloat16), (((1,), (0,)), ((), ())),
        preferred_element_type=jnp.float32)
    rank_rows = excl + cnt_ref[...]
    rank_sel = jnp.sum(M * rank_rows, axis=1, keepdims=True)

    eid_ref[0] = sel_i
    rank_ref[0] = rank_sel.astype(jnp.int32)
    wpair_ref[0] = sel_w

    cnt_ref[...] += jnp.sum(M, axis=0, keepdims=True)

    @pl.when(kk == 0)
    def _():
        ex = jnp.exp(logits - m1)
        probs = ex / jnp.sum(ex, axis=1, keepdims=True)
        acc_ref[...] += jnp.sum(probs, axis=0, keepdims=True)

    @pl.when(last)
    def _():
        pm = acc_ref[...] / N
        laux_ref[...] = jnp.sum(pm * pm, keepdims=True).reshape(1, 1) * E

        cntv = cnt_ref[...]
        blk = jnp.floor((cntv + (BLK - 1)) / BLK)
        padded = blk * BLK
        ri = jax.lax.broadcasted_iota(jnp.int32, (E, E), 0)
        cj = jax.lax.broadcasted_iota(jnp.int32, (E, E), 1)
        upt = (ri <= cj).astype(jnp.float32)
        base_incl = jax.lax.dot_general(
            padded, upt, (((1,), (0,)), ((), ())),
            preferred_element_type=jnp.float32, precision=jax.lax.Precision.HIGHEST)
        base = base_incl - padded
        blk_incl = jax.lax.dot_general(
            blk, upt, (((1,), (0,)), ((), ())),
            preferred_element_type=jnp.float32, precision=jax.lax.Precision.HIGHEST)
        blkoff = blk_incl - blk
        base16_ref[...] = jnp.pad(base.astype(jnp.int32), ((0, 0), (0, 16 - E)))

        bvec = jax.lax.broadcasted_iota(jnp.int32, (1, GRID_C), 1)
        lane_e = jax.lax.broadcasted_iota(jnp.int32, (1, E), 1)
        e_of_b = jnp.zeros((1, GRID_C), jnp.int32)
        for e in range(E - 1):
            incl_e = jnp.sum(
                jnp.where(lane_e == e, blk_incl, 0.0), axis=1,
                keepdims=True).astype(jnp.int32)
            e_of_b = e_of_b + (bvec >= incl_e).astype(jnp.int32)
        nused = jnp.sum(
            jnp.where(lane_e == E - 1, blk_incl, 0.0), axis=1,
            keepdims=True).astype(jnp.int32)
        validv = (bvec < nused).astype(jnp.int32)
        valid_ref[...] = validv
        xsidx_ref[...] = jnp.minimum(e_of_b, E - 1)
        outidx_ref[...] = jnp.where(validv == 1, bvec, TRASH)


def _router(flat, gate_W, gate_b, stril):
    nt = N // _RT
    (eid, rank, wpair, base16, valid, xsidx, outidx,
     laux) = pl.pallas_call(
        _router_body,
        grid=(TOPK, nt),
        in_specs=[
            pl.BlockSpec((_RT, D), lambda k, t: (t, 0)),
            pl.BlockSpec((E, D), lambda k, t: (0, 0)),
            pl.BlockSpec((1, E), lambda k, t: (0, 0)),
            pl.BlockSpec((_RT, _RT), lambda k, t: (0, 0)),
        ],
        out_specs=[
            pl.BlockSpec((1, _RT, 1), lambda k, t: (k * (N // _RT) + t, 0, 0)),
            pl.BlockSpec((1, _RT, 1), lambda k, t: (k * (N // _RT) + t, 0, 0)),
            pl.BlockSpec((1, _RT, 1), lambda k, t: (k * (N // _RT) + t, 0, 0)),
            pl.BlockSpec((1, 16), lambda k, t: (0, 0)),
            pl.BlockSpec((1, GRID_C), lambda k, t: (0, 0)),
            pl.BlockSpec((1, GRID_C), lambda k, t: (0, 0)),
            pl.BlockSpec((1, GRID_C), lambda k, t: (0, 0)),
            pl.BlockSpec((1, 1), lambda k, t: (0, 0)),
        ],
        out_shape=[
            jax.ShapeDtypeStruct((TOPK * nt, _RT, 1), jnp.int32),
            jax.ShapeDtypeStruct((TOPK * nt, _RT, 1), jnp.int32),
            jax.ShapeDtypeStruct((TOPK * nt, _RT, 1), jnp.float32),
            jax.ShapeDtypeStruct((1, 16), jnp.int32),
            jax.ShapeDtypeStruct((1, GRID_C), jnp.int32),
            jax.ShapeDtypeStruct((1, GRID_C), jnp.int32),
            jax.ShapeDtypeStruct((1, GRID_C), jnp.int32),
            jax.ShapeDtypeStruct((1, 1), jnp.float32),
        ],
        scratch_shapes=[pltpu.VMEM((1, E), jnp.float32),
                        pltpu.VMEM((1, E), jnp.float32)],
        compiler_params=pltpu.CompilerParams(
            dimension_semantics=("arbitrary", "arbitrary"),
        ),
    )(flat, gate_W, gate_b.reshape(1, E), stril)
    return (eid.reshape(P), rank.reshape(P), wpair.reshape(P),
            base16.reshape(16), valid.reshape(GRID_C),
            xsidx.reshape(GRID_C), outidx.reshape(GRID_C), laux)


def _dispatch_body(x_hbm, eid_hbm, rank_hbm, w_hbm, base_hbm,
                   xs_hbm, wrow_hbm, pos_hbm,
                   base_v, eid_v, rank_v, w_v, pos_v,
                   pos2_v, tok2_v, rows_v, wbuf_v, sem):
    wid = lax.axis_index("s") * 2 + lax.axis_index("c")
    pbase = wid * (P // NW)

    pltpu.sync_copy(base_hbm, base_v)
    pltpu.sync_copy(eid_hbm.at[pl.ds(pbase, P // NW)], eid_v)
    pltpu.sync_copy(rank_hbm.at[pl.ds(pbase, P // NW)], rank_v)
    pltpu.sync_copy(w_hbm.at[pl.ds(pbase, P // NW)], w_v)

    def posloop(ci, carry):
        off = ci * 16
        ev = eid_v[pl.ds(off, 16)]
        rv = rank_v[pl.ds(off, 16)]
        bv = plsc.load_gather(base_v, [ev])
        posv = bv + rv
        pos_v[pl.ds(off, 16)] = posv
        pos2_v[ci, :] = posv
        tok2_v[ci, :] = (pbase + off + lax.iota(jnp.int32, 16)) & (N - 1)
        return carry

    lax.fori_loop(0, P // NW // 16, posloop, 0, unroll=False)
    pltpu.sync_copy(pos_v, pos_hbm.at[pl.ds(pbase, P // NW)])

    def chunk(ci, carry):
        off = ci * 16
        pltpu.async_copy(x_hbm.at[tok2_v.at[ci]], rows_v, sem).wait()
        pltpu.async_copy(rows_v, xs_hbm.at[pos2_v.at[ci]], sem).wait()
        wv = w_v[pl.ds(off, 16)]
        for j in range(16):
            wbuf_v[j, pl.ds(0, 16)] = wv[jnp.full((16,), j, jnp.int32)]
        pltpu.async_copy(wbuf_v, wrow_hbm.at[pos2_v.at[ci]], sem).wait()
        return carry

    lax.fori_loop(0, P // NW // 16, chunk, 0, unroll=False)


def _dispatch(x2d, eid, rank, wpair, base16):
    mesh = plsc.VectorSubcoreMesh(core_axis_name="c", subcore_axis_name="s")
    kfn = pl.kernel(
        _dispatch_body,
        compiler_params=pltpu.CompilerParams(needs_layout_passes=False),
        out_type=[
            jax.ShapeDtypeStruct((PMAX, D), jnp.float32),
            jax.ShapeDtypeStruct((PMAX, 128), jnp.float32),
            jax.ShapeDtypeStruct((P,), jnp.int32),
        ],
        mesh=mesh,
        scratch_types=[
            pltpu.VMEM((16,), jnp.int32),
            pltpu.VMEM((P // NW,), jnp.int32),
            pltpu.VMEM((P // NW,), jnp.int32),
            pltpu.VMEM((P // NW,), jnp.float32),
            pltpu.VMEM((P // NW,), jnp.int32),
            pltpu.VMEM((P // NW // 16, 16), jnp.int32),
            pltpu.VMEM((P // NW // 16, 16), jnp.int32),
            pltpu.VMEM((16, D), jnp.float32),
            pltpu.VMEM((16, 128), jnp.float32),
            pltpu.SemaphoreType.DMA,
        ],
    )
    return kfn(x2d, eid, rank, wpair, base16)


def _gffn_body(valid_ref, eofb_ref, outidx_ref,
               xs_ref, w1_ref, b1_ref, w2_ref, b2_ref, wrow_ref, out_ref):
    g = pl.program_id(0)
    valid = valid_ref[g] == 1

    @pl.when(valid)
    def _():
        xb = xs_ref[...].astype(jnp.bfloat16)
        hpre = jax.lax.dot_general(
            xb, w1_ref[0], (((1,), (1,)), ((), ())),
            preferred_element_type=jnp.float32) + b1_ref[0]
        hact = hpre * 0.5 * (1.0 + jax.lax.erf(hpre * _SQRT_HALF))
        contrib = jax.lax.dot_general(
            hact, w2_ref[0], (((1,), (1,)), ((), ())),
            preferred_element_type=jnp.float32,
            precision=jax.lax.Precision.DEFAULT) + b2_ref[0]
        wcol = wrow_ref[...][:, 0:1]
        out_ref[...] = contrib * wcol

    @pl.when(~valid)
    def _():
        out_ref[...] = jnp.zeros_like(out_ref)


def _gffn(xs, wrow, valid, eofb, outidx, W1b, b1, W2b, b2):
    grid_spec = pltpu.PrefetchScalarGridSpec(
        num_scalar_prefetch=3,
        grid=(NBLK,),
        in_specs=[
            pl.BlockSpec((BLK, D), lambda g, v, eb, oi: (g, 0)),
            pl.BlockSpec((1, H, D), lambda g, v, eb, oi: (eb[g], 0, 0)),
            pl.BlockSpec((1, 1, H), lambda g, v, eb, oi: (eb[g], 0, 0)),
            pl.BlockSpec((1, D, H), lambda g, v, eb, oi: (eb[g], 0, 0)),
            pl.BlockSpec((1, 1, D), lambda g, v, eb, oi: (eb[g], 0, 0)),
            pl.BlockSpec((BLK, 128), lambda g, v, eb, oi: (g, 0)),
        ],
        out_specs=pl.BlockSpec((BLK, D), lambda g, v, eb, oi: (oi[g], 0)),
    )
    return pl.pallas_call(
        _gffn_body,
        grid_spec=grid_spec,
        out_shape=jax.ShapeDtypeStruct(((NBLK + 1) * BLK, D), jnp.float32),
        compiler_params=pltpu.CompilerParams(
            dimension_semantics=("arbitrary",),
        ),
    )(valid, eofb, outidx, xs, W1b, b1.reshape(E, 1, H), W2b,
      b2.reshape(E, 1, D), wrow)


def _combine_body(ys_hbm, p1_hbm, p2_hbm, out_hbm, p1_v, p2_v, buf_v, buf2_v,
                  sem):
    wid = lax.axis_index("s") * 2 + lax.axis_index("c")
    tbase = wid * (N // NW)

    pltpu.sync_copy(p1_hbm.at[pl.ds(tbase, N // NW)], p1_v)
    pltpu.sync_copy(p2_hbm.at[pl.ds(tbase, N // NW)], p2_v)

    def chunk(ci, carry):
        off = ci * 16
        i1 = p1_v[pl.ds(off, 16)]
        i2 = p2_v[pl.ds(off, 16)]
        cp1 = pltpu.async_copy(ys_hbm.at[i1], buf_v, sem)
        cp2 = pltpu.async_copy(ys_hbm.at[i2], buf2_v, sem)
        cp1.wait()
        cp2.wait()

        def addrow(j, carry2):
            for seg in range(D // 16):
                s = pl.ds(seg * 16, 16)
                buf_v[j, s] = buf_v[j, s] + buf2_v[j, s]
            return carry2

        lax.fori_loop(0, 16, addrow, 0, unroll=False)
        pltpu.sync_copy(buf_v, out_hbm.at[pl.ds(tbase + off, 16)])
        return carry

    lax.fori_loop(0, N // NW // 16, chunk, 0, unroll=False)


def _combine(ys, pos1, pos2):
    mesh = plsc.VectorSubcoreMesh(core_axis_name="c", subcore_axis_name="s")
    kfn = pl.kernel(
        _combine_body,
        out_type=jax.ShapeDtypeStruct((N, D), jnp.float32),
        mesh=mesh,
        scratch_types=[
            pltpu.VMEM((N // NW,), jnp.int32),
            pltpu.VMEM((N // NW,), jnp.int32),
            pltpu.VMEM((16, D), jnp.float32),
            pltpu.VMEM((16, D), jnp.float32),
            pltpu.SemaphoreType.DMA,
        ],
    )
    return kfn(ys, pos1, pos2)


def kernel(x, gate_W, gate_b, W1, b1, W2, b2):
    flat = x.reshape(N, D)
    W1b = W1.astype(jnp.bfloat16)
    W2b = W2
    ii = jnp.arange(_RT, dtype=jnp.int32)
    stril = (ii[None, :] < ii[:, None]).astype(jnp.bfloat16)
    (eid, rank, wpair, base16, valid, xsidx, outidx,
     laux) = _router(flat, gate_W, gate_b, stril)
    xs, wrow, pos = _dispatch(flat, eid, rank, wpair, base16)
    ys = _gffn(xs, wrow, valid, xsidx, outidx, W1b, b1, W2b, b2)
    out_flat = _combine(ys, pos[:N], pos[N:])
    return out_flat.reshape(B, S, D), laux.reshape(())
